# baseline (speedup 1.0000x reference)
import functools

import jax
import jax.numpy as jnp
from jax import lax
from jax.experimental import pallas as pl
from jax.experimental.pallas import tpu as pltpu
from jax.experimental.pallas import tpu_sc as plsc

N = 10000
E = 320000
IN_DIM = 128
HID = 64

NC = 2
NS = 16
NW = NC * NS
EPW = E // NW
CH = 200
NCHUNK = EPW // CH
NP = 10240
RPW = NP // NS

_mesh = plsc.VectorSubcoreMesh(core_axis_name="c", subcore_axis_name="s",
                               num_cores=NC)


NBUF = 4


def _sc_body(with_deg, *refs):
    if with_deg:
        (t_hbm, ei_hbm, z64_hbm, z8_hbm, ones_hbm,
         agg_out, deg_out, src_v, dst_v, *rest) = refs
        bufs = rest[:NBUF]
        ones_v, acc, degacc = rest[NBUF:NBUF + 3]
        sems = rest[NBUF + 3:]
        sg = sems[:NBUF]
        ss = sems[NBUF:2 * NBUF]
        sd = sems[2 * NBUF]
    else:
        (t_hbm, ei_hbm, z64_hbm,
         agg_out, src_v, dst_v, *rest) = refs
        bufs = rest[:NBUF]
        acc = rest[NBUF]
        sems = rest[NBUF + 1:]
        sg = sems[:NBUF]
        ss = sems[NBUF:2 * NBUF]
        sd = ones_v = degacc = None

    c = lax.axis_index("c")
    s = lax.axis_index("s")
    wid = c * NS + s

    pltpu.sync_copy(ei_hbm.at[0].at[wid], src_v)
    pltpu.sync_copy(ei_hbm.at[1].at[wid], dst_v)
    pltpu.sync_copy(z64_hbm.at[pl.ds(s * RPW, RPW)],
                    acc.at[pl.ds(s * RPW, RPW)])
    if with_deg:
        pltpu.sync_copy(z8_hbm.at[pl.ds(s * RPW, RPW)],
                        degacc.at[pl.ds(s * RPW, RPW)])
        pltpu.sync_copy(ones_hbm, ones_v)
    plsc.subcore_barrier()

    def gather(j, buf, sem):
        pltpu.async_copy(t_hbm.at[src_v.at[j]], buf, sem)

    def gwait(buf, sem):
        pltpu.make_async_copy(t_hbm.at[src_v.at[0]], buf, sem).wait()

    def scatter(j, buf, sem):
        pltpu.async_copy(buf, acc.at[dst_v.at[j]], sem, add=True)

    def swait(buf, sem):
        pltpu.make_async_copy(buf, acc.at[dst_v.at[0]], sem).wait()

    def deg_scatter(j):
        pltpu.async_copy(ones_v, degacc.at[dst_v.at[j]], sd, add=True)

    def deg_wait():
        pltpu.make_async_copy(ones_v, degacc.at[dst_v.at[0]], sd).wait()

    FULL = NCHUNK // NBUF
    REM = NCHUNK % NBUF

    for b in range(NBUF):
        gather(b, bufs[b], sg[b])

    def round_(i, carry):
        for b in range(NBUF):
            j = i * NBUF + b
            gwait(bufs[b], sg[b])
            scatter(j, bufs[b], ss[b])
            if with_deg:
                deg_scatter(j)

                @pl.when(j >= NBUF)
                def _():
                    deg_wait()

            nxt = j + NBUF

            @pl.when(nxt < NCHUNK)
            def _():
                swait(bufs[b], ss[b])
                gather(nxt, bufs[b], sg[b])
        return carry

    lax.fori_loop(0, FULL, round_, 0)

    for b in range(REM):
        j = FULL * NBUF + b
        gwait(bufs[b], sg[b])
        scatter(j, bufs[b], ss[b])
        if with_deg:
            deg_scatter(j)

    for b in range(NBUF):
        swait(bufs[b], ss[b])
    if with_deg:
        for _ in range(min(NBUF + REM, NCHUNK)):
            deg_wait()
    plsc.subcore_barrier()

    pltpu.sync_copy(acc.at[pl.ds(s * RPW, RPW)],
                    agg_out.at[c].at[pl.ds(s * RPW, RPW)])
    if with_deg:
        pltpu.sync_copy(degacc.at[pl.ds(s * RPW, RPW)],
                        deg_out.at[c].at[pl.ds(s * RPW, RPW)])


def _sc_scratch_common():
    return [
        pltpu.VMEM((NCHUNK, CH), jnp.int32),
        pltpu.VMEM((NCHUNK, CH), jnp.int32),
    ] + [pltpu.VMEM((CH, HID), jnp.float32) for _ in range(NBUF)]


_sc_params = pltpu.CompilerParams(use_tc_tiling_on_sc=False)

_sc_agg_deg = pl.kernel(
    functools.partial(_sc_body, True),
    out_type=(jax.ShapeDtypeStruct((NC, NP, HID), jnp.float32),
              jax.ShapeDtypeStruct((NC, NP, 8), jnp.float32)),
    mesh=_mesh,
    compiler_params=_sc_params,
    scratch_types=_sc_scratch_common() + [
        pltpu.VMEM((CH, 8), jnp.float32),
        pltpu.VMEM_SHARED((NP, HID), jnp.float32),
        pltpu.VMEM_SHARED((NP, 8), jnp.float32),
    ] + [pltpu.SemaphoreType.DMA] * (2 * NBUF + 1),
)

_sc_agg = pl.kernel(
    functools.partial(_sc_body, False),
    out_type=jax.ShapeDtypeStruct((NC, NP, HID), jnp.float32),
    mesh=_mesh,
    compiler_params=_sc_params,
    scratch_types=_sc_scratch_common() + [
        pltpu.VMEM_SHARED((NP, HID), jnp.float32),
    ] + [pltpu.SemaphoreType.DMA] * (2 * NBUF),
)


BR = 2000
GRID = N // BR


def _proj1_body(x_ref, wl_ref, wr_ref, b_ref, t_ref, r_ref):
    xb = x_ref[...]
    t_ref[...] = jnp.dot(xb, wl_ref[...], preferred_element_type=jnp.float32)
    r_ref[...] = (jnp.dot(xb, wr_ref[...], preferred_element_type=jnp.float32)
                  + b_ref[...])


def _mid_body(agg_ref, deg_ref, r_ref, wl_ref, wr_ref, b_ref, t_ref, r2_ref):
    agg = agg_ref[0] + agg_ref[1]
    deg = deg_ref[0, :, 0:1] + deg_ref[1, :, 0:1]
    h = jnp.maximum(agg / jnp.maximum(deg, 1.0) + r_ref[...], 0.0)
    t_ref[...] = jnp.dot(h, wl_ref[...], preferred_element_type=jnp.float32)
    r2_ref[...] = (jnp.dot(h, wr_ref[...], preferred_element_type=jnp.float32)
                   + b_ref[...])


def _final_body(agg_ref, deg_ref, r_ref, wlin_ref, blin_ref, o_ref):
    agg = agg_ref[0] + agg_ref[1]
    deg = deg_ref[0, :, 0:1] + deg_ref[1, :, 0:1]
    h = jnp.maximum(agg / jnp.maximum(deg, 1.0) + r_ref[...], 0.0)
    o_ref[...] = (jnp.sum(h * wlin_ref[...], axis=1) + blin_ref[0]).reshape(
        1, 1, BR)


def _row_spec(d):
    return pl.BlockSpec((BR, d), lambda i: (i, 0))


def _dup_spec(d):
    return pl.BlockSpec((NC, BR, d), lambda i: (0, i, 0))


def _full_spec(a, b):
    return pl.BlockSpec((a, b), lambda i: (0, 0))


_proj1 = pl.pallas_call(
    _proj1_body,
    grid=(GRID,),
    in_specs=[_row_spec(IN_DIM), _full_spec(IN_DIM, HID),
              _full_spec(IN_DIM, HID), _full_spec(1, HID)],
    out_specs=[_row_spec(HID), _row_spec(HID)],
    out_shape=(jax.ShapeDtypeStruct((N, HID), jnp.float32),
               jax.ShapeDtypeStruct((N, HID), jnp.float32)),
)

_mid = pl.pallas_call(
    _mid_body,
    grid=(GRID,),
    in_specs=[_dup_spec(HID), _dup_spec(8), _row_spec(HID),
              _full_spec(HID, HID), _full_spec(HID, HID), _full_spec(1, HID)],
    out_specs=[_row_spec(HID), _row_spec(HID)],
    out_shape=(jax.ShapeDtypeStruct((N, HID), jnp.float32),
               jax.ShapeDtypeStruct((N, HID), jnp.float32)),
)

_final = pl.pallas_call(
    _final_body,
    grid=(GRID,),
    in_specs=[_dup_spec(HID), _dup_spec(8), _row_spec(HID),
              _full_spec(1, HID), pl.BlockSpec(memory_space=pltpu.SMEM)],
    out_specs=pl.BlockSpec((1, 1, BR), lambda i: (i, 0, 0)),
    out_shape=jax.ShapeDtypeStruct((GRID, 1, BR), jnp.float32),
)


def kernel(x, edge_index, W1_l, b1_l, W1_r, W2_l, b2_l, W2_r, W_lin, b_lin):
    ei = edge_index.astype(jnp.int32).reshape(2, NW, NCHUNK, CH)

    z64 = jnp.zeros((NP, HID), jnp.float32)
    z8 = jnp.zeros((NP, 8), jnp.float32)
    ones8 = jnp.ones((CH, 8), jnp.float32)

    t1, r1 = _proj1(x, W1_l.T, W1_r.T, b1_l.reshape(1, HID))
    agg1, deg = _sc_agg_deg(t1, ei, z64, z8, ones8)
    t2, r2 = _mid(agg1, deg, r1, W2_l.T, W2_r.T, b2_l.reshape(1, HID))
    agg2 = _sc_agg(t2, ei, z64)
    out = _final(agg2, deg, r2, W_lin, b_lin)
    return out.reshape(N)


# speedup vs baseline: 1.0220x; 1.0220x over previous
import functools

import jax
import jax.numpy as jnp
from jax import lax
from jax.experimental import pallas as pl
from jax.experimental.pallas import tpu as pltpu
from jax.experimental.pallas import tpu_sc as plsc

N = 10000
E = 320000
IN_DIM = 128
HID = 64

NC = 2
NS = 16
NW = NC * NS
EPW = E // NW
CH = 80
NCHUNK = EPW // CH
NP = 10240
RPW = NP // NS

_mesh = plsc.VectorSubcoreMesh(core_axis_name="c", subcore_axis_name="s",
                               num_cores=NC)


NBUF = 6


def _sc_body(with_deg, *refs):
    if with_deg:
        (t_hbm, ei_hbm, z64_hbm, z8_hbm, ones_hbm,
         agg_out, deg_out, src_v, dst_v, *rest) = refs
        bufs = rest[:NBUF]
        ones_v, acc, degacc = rest[NBUF:NBUF + 3]
        sems = rest[NBUF + 3:]
        sg = sems[:NBUF]
        ss = sems[NBUF:2 * NBUF]
        sd = sems[2 * NBUF]
    else:
        (t_hbm, ei_hbm, z64_hbm,
         agg_out, src_v, dst_v, *rest) = refs
        bufs = rest[:NBUF]
        acc = rest[NBUF]
        sems = rest[NBUF + 1:]
        sg = sems[:NBUF]
        ss = sems[NBUF:2 * NBUF]
        sd = ones_v = degacc = None

    c = lax.axis_index("c")
    s = lax.axis_index("s")
    wid = c * NS + s

    pltpu.sync_copy(ei_hbm.at[0].at[wid], src_v)
    pltpu.sync_copy(ei_hbm.at[1].at[wid], dst_v)
    pltpu.sync_copy(z64_hbm.at[pl.ds(s * RPW, RPW)],
                    acc.at[pl.ds(s * RPW, RPW)])
    if with_deg:
        pltpu.sync_copy(z8_hbm.at[pl.ds(s * RPW, RPW)],
                        degacc.at[pl.ds(s * RPW, RPW)])
        pltpu.sync_copy(ones_hbm, ones_v)
    plsc.subcore_barrier()

    def gather(j, buf, sem):
        pltpu.async_copy(t_hbm.at[src_v.at[j]], buf, sem)

    def gwait(buf, sem):
        pltpu.make_async_copy(t_hbm.at[src_v.at[0]], buf, sem).wait()

    def scatter(j, buf, sem):
        pltpu.async_copy(buf, acc.at[dst_v.at[j]], sem, add=True)

    def swait(buf, sem):
        pltpu.make_async_copy(buf, acc.at[dst_v.at[0]], sem).wait()

    def deg_scatter(j):
        pltpu.async_copy(ones_v, degacc.at[dst_v.at[j]], sd, add=True)

    def deg_wait():
        pltpu.make_async_copy(ones_v, degacc.at[dst_v.at[0]], sd).wait()

    FULL = NCHUNK // NBUF
    REM = NCHUNK % NBUF

    for b in range(NBUF):
        gather(b, bufs[b], sg[b])

    def round_(i, carry):
        for b in range(NBUF):
            j = i * NBUF + b
            gwait(bufs[b], sg[b])
            scatter(j, bufs[b], ss[b])
            if with_deg:
                deg_scatter(j)

                @pl.when(j >= NBUF)
                def _():
                    deg_wait()

            nxt = j + NBUF

            @pl.when(nxt < NCHUNK)
            def _():
                swait(bufs[b], ss[b])
                gather(nxt, bufs[b], sg[b])
        return carry

    lax.fori_loop(0, FULL, round_, 0)

    for b in range(REM):
        j = FULL * NBUF + b
        gwait(bufs[b], sg[b])
        scatter(j, bufs[b], ss[b])
        if with_deg:
            deg_scatter(j)

    for b in range(NBUF):
        swait(bufs[b], ss[b])
    if with_deg:
        for _ in range(min(NBUF + REM, NCHUNK)):
            deg_wait()
    plsc.subcore_barrier()

    pltpu.sync_copy(acc.at[pl.ds(s * RPW, RPW)],
                    agg_out.at[c].at[pl.ds(s * RPW, RPW)])
    if with_deg:
        pltpu.sync_copy(degacc.at[pl.ds(s * RPW, RPW)],
                        deg_out.at[c].at[pl.ds(s * RPW, RPW)])


def _sc_scratch_common():
    return [
        pltpu.VMEM((NCHUNK, CH), jnp.int32),
        pltpu.VMEM((NCHUNK, CH), jnp.int32),
    ] + [pltpu.VMEM((CH, HID), jnp.float32) for _ in range(NBUF)]


_sc_params = pltpu.CompilerParams(use_tc_tiling_on_sc=False)

_sc_agg_deg = pl.kernel(
    functools.partial(_sc_body, True),
    out_type=(jax.ShapeDtypeStruct((NC, NP, HID), jnp.float32),
              jax.ShapeDtypeStruct((NC, NP, 8), jnp.float32)),
    mesh=_mesh,
    compiler_params=_sc_params,
    scratch_types=_sc_scratch_common() + [
        pltpu.VMEM((CH, 8), jnp.float32),
        pltpu.VMEM_SHARED((NP, HID), jnp.float32),
        pltpu.VMEM_SHARED((NP, 8), jnp.float32),
    ] + [pltpu.SemaphoreType.DMA] * (2 * NBUF + 1),
)

_sc_agg = pl.kernel(
    functools.partial(_sc_body, False),
    out_type=jax.ShapeDtypeStruct((NC, NP, HID), jnp.float32),
    mesh=_mesh,
    compiler_params=_sc_params,
    scratch_types=_sc_scratch_common() + [
        pltpu.VMEM_SHARED((NP, HID), jnp.float32),
    ] + [pltpu.SemaphoreType.DMA] * (2 * NBUF),
)


BR = 2000
GRID = N // BR


def _dot_nt(a, w):
    return lax.dot_general(a, w, (((1,), (1,)), ((), ())),
                           preferred_element_type=jnp.float32)


def _proj1_body(x_ref, wl_ref, wr_ref, b_ref, t_ref, r_ref):
    xb = x_ref[...]
    t_ref[...] = _dot_nt(xb, wl_ref[...])
    r_ref[...] = _dot_nt(xb, wr_ref[...]) + b_ref[...]


def _mid_body(agg_ref, deg_ref, r_ref, wl_ref, wr_ref, b_ref, t_ref, r2_ref):
    agg = agg_ref[0] + agg_ref[1]
    deg = deg_ref[0, :, 0:1] + deg_ref[1, :, 0:1]
    h = jnp.maximum(agg / jnp.maximum(deg, 1.0) + r_ref[...], 0.0)
    t_ref[...] = _dot_nt(h, wl_ref[...])
    r2_ref[...] = _dot_nt(h, wr_ref[...]) + b_ref[...]


def _final_body(agg_ref, deg_ref, r_ref, wlin_ref, blin_ref, o_ref):
    agg = agg_ref[0] + agg_ref[1]
    deg = deg_ref[0, :, 0:1] + deg_ref[1, :, 0:1]
    h = jnp.maximum(agg / jnp.maximum(deg, 1.0) + r_ref[...], 0.0)
    o_ref[...] = (jnp.sum(h * wlin_ref[...], axis=1) + blin_ref[0]).reshape(
        1, 1, BR)


def _row_spec(d):
    return pl.BlockSpec((BR, d), lambda i: (i, 0))


def _dup_spec(d):
    return pl.BlockSpec((NC, BR, d), lambda i: (0, i, 0))


def _full_spec(a, b):
    return pl.BlockSpec((a, b), lambda i: (0, 0))


_proj1 = pl.pallas_call(
    _proj1_body,
    grid=(GRID,),
    in_specs=[_row_spec(IN_DIM), _full_spec(HID, IN_DIM),
              _full_spec(HID, IN_DIM), _full_spec(1, HID)],
    out_specs=[_row_spec(HID), _row_spec(HID)],
    out_shape=(jax.ShapeDtypeStruct((N, HID), jnp.float32),
               jax.ShapeDtypeStruct((N, HID), jnp.float32)),
)

_mid = pl.pallas_call(
    _mid_body,
    grid=(GRID,),
    in_specs=[_dup_spec(HID), _dup_spec(8), _row_spec(HID),
              _full_spec(HID, HID), _full_spec(HID, HID), _full_spec(1, HID)],
    out_specs=[_row_spec(HID), _row_spec(HID)],
    out_shape=(jax.ShapeDtypeStruct((N, HID), jnp.float32),
               jax.ShapeDtypeStruct((N, HID), jnp.float32)),
)

_final = pl.pallas_call(
    _final_body,
    grid=(GRID,),
    in_specs=[_dup_spec(HID), _dup_spec(8), _row_spec(HID),
              _full_spec(1, HID), pl.BlockSpec(memory_space=pltpu.SMEM)],
    out_specs=pl.BlockSpec((1, 1, BR), lambda i: (i, 0, 0)),
    out_shape=jax.ShapeDtypeStruct((GRID, 1, BR), jnp.float32),
)


def kernel(x, edge_index, W1_l, b1_l, W1_r, W2_l, b2_l, W2_r, W_lin, b_lin):
    ei = edge_index.astype(jnp.int32).reshape(2, NW, NCHUNK, CH)

    z64 = jnp.zeros((NP, HID), jnp.float32)
    z8 = jnp.zeros((NP, 8), jnp.float32)
    ones8 = jnp.ones((CH, 8), jnp.float32)

    t1, r1 = _proj1(x, W1_l, W1_r, b1_l.reshape(1, HID))
    agg1, deg = _sc_agg_deg(t1, ei, z64, z8, ones8)
    t2, r2 = _mid(agg1, deg, r1, W2_l, W2_r, b2_l.reshape(1, HID))
    agg2 = _sc_agg(t2, ei, z64)
    out = _final(agg2, deg, r2, W_lin, b_lin)
    return out.reshape(N)
